# Initial kernel scaffold; baseline (speedup 1.0000x reference)
import jax
import jax.numpy as jnp
from jax.experimental import pallas as pl


def kernel(feat_x, feat_y):
    raise NotImplementedError("write your pallas kernel here")


# speedup vs baseline: 24.5083x; 24.5083x over previous
import functools

import jax
import jax.numpy as jnp
from jax.experimental import pallas as pl

_TAU = 0.2
_K = 15


def _normalize_rows(x):
    ss = jnp.sum(x * x, axis=-1, keepdims=True)
    n = jnp.maximum(jnp.sqrt(ss), 1e-12)
    return x / n


def _normalize_kernel(x_ref, o_ref):
    o_ref[...] = _normalize_rows(x_ref[...])


def _simtopk_kernel(x_ref, yn_ref, o_ref):
    x = _normalize_rows(x_ref[0])
    y = yn_ref[0]
    s = jax.lax.dot_general(
        x, y, (((1,), (1,)), ((), ())),
        preferred_element_type=jnp.float32,
    ) / _TAU
    m = jnp.max(s, axis=-1, keepdims=True)
    rowmax = m
    for _ in range(_K - 1):
        m = jnp.max(jnp.where(s < m, s, -jnp.inf), axis=-1, keepdims=True)
    e = jnp.where(s >= m, jnp.exp(s - rowmax), 0.0)
    o_ref[0] = e / jnp.sum(e, axis=-1, keepdims=True)


def kernel(feat_x, feat_y):
    B, Nx, C = feat_x.shape
    Ny = feat_y.shape[1]
    BX = 256

    yn = pl.pallas_call(
        _normalize_kernel,
        grid=(B,),
        in_specs=[pl.BlockSpec((1, Ny, C), lambda b: (b, 0, 0))],
        out_specs=pl.BlockSpec((1, Ny, C), lambda b: (b, 0, 0)),
        out_shape=jax.ShapeDtypeStruct((B, Ny, C), jnp.float32),
    )(feat_y)

    out = pl.pallas_call(
        _simtopk_kernel,
        grid=(B, Nx // BX),
        in_specs=[
            pl.BlockSpec((1, BX, C), lambda b, i: (b, i, 0)),
            pl.BlockSpec((1, Ny, C), lambda b, i: (b, 0, 0)),
        ],
        out_specs=pl.BlockSpec((1, BX, Ny), lambda b, i: (b, i, 0)),
        out_shape=jax.ShapeDtypeStruct((B, Nx, Ny), jnp.float32),
    )(feat_x, yn)
    return out


# speedup vs baseline: 25.2923x; 1.0320x over previous
import math

import jax
import jax.numpy as jnp
from jax.experimental import pallas as pl

_TAU = 0.2
_K = 15
_SCALE = math.log2(math.e) / _TAU


def _normalize_rows(x, scale=1.0):
    ss = jnp.sum(x * x, axis=-1, keepdims=True)
    n = jnp.maximum(jnp.sqrt(ss), 1e-12)
    return x * (scale / n)


def _normalize_y_kernel(x_ref, o_ref):
    o_ref[...] = _normalize_rows(x_ref[...], _SCALE)


def _simtopk_kernel(x_ref, yn_ref, o_ref):
    x = _normalize_rows(x_ref[0])
    y = yn_ref[0]
    s = jax.lax.dot_general(
        x, y, (((1,), (1,)), ((), ())),
        preferred_element_type=jnp.float32,
    )
    m = jnp.max(s, axis=-1, keepdims=True)
    for _ in range(_K - 1):
        m = jnp.max(jnp.where(s < m, s, -jnp.inf), axis=-1, keepdims=True)
    e = jnp.where(s >= m, jnp.exp2(s), 0.0)
    o_ref[0] = e / jnp.sum(e, axis=-1, keepdims=True)


def kernel(feat_x, feat_y):
    B, Nx, C = feat_x.shape
    Ny = feat_y.shape[1]
    BX = 256

    yn = pl.pallas_call(
        _normalize_y_kernel,
        grid=(B,),
        in_specs=[pl.BlockSpec((1, Ny, C), lambda b: (b, 0, 0))],
        out_specs=pl.BlockSpec((1, Ny, C), lambda b: (b, 0, 0)),
        out_shape=jax.ShapeDtypeStruct((B, Ny, C), jnp.float32),
    )(feat_y)

    out = pl.pallas_call(
        _simtopk_kernel,
        grid=(B, Nx // BX),
        in_specs=[
            pl.BlockSpec((1, BX, C), lambda b, i: (b, i, 0)),
            pl.BlockSpec((1, Ny, C), lambda b, i: (b, 0, 0)),
        ],
        out_specs=pl.BlockSpec((1, BX, Ny), lambda b, i: (b, i, 0)),
        out_shape=jax.ShapeDtypeStruct((B, Nx, Ny), jnp.float32),
    )(feat_x, yn)
    return out
